# Initial kernel scaffold; baseline (speedup 1.0000x reference)
import jax
import jax.numpy as jnp
from jax.experimental import pallas as pl


def kernel(x, w0, b0, w1, b1, wl, bl):
    raise NotImplementedError("write your pallas kernel here")


# speedup vs baseline: 1.7063x; 1.7063x over previous
import functools

import jax
import jax.numpy as jnp
from jax.experimental import pallas as pl
from jax.experimental.pallas import tpu as pltpu

LANE = 128
POOL = 32
KSZ = 32


def _toeplitz(w):
    cout, cin, k = w.shape
    s = jnp.arange(2 * POOL)[None, :]
    r = jnp.arange(POOL)[:, None]
    d = s - r
    mask = (d >= 0) & (d < k)
    g = w[:, :, jnp.clip(d, 0, k - 1)]
    g = jnp.where(mask[None, None], g, 0.0)
    g = g.transpose(0, 2, 1, 3)
    return g.reshape(cout * POOL, cin * 2 * POOL)


def _phase_major(x, j_out, jpad):
    b, cin, l = x.shape
    need = POOL * (j_out + 2)
    xpad = jnp.pad(x, ((0, 0), (0, 0), (0, max(0, need - l))))
    a = xpad[..., :POOL * j_out].reshape(b, cin, j_out, POOL).transpose(0, 1, 3, 2)
    c = xpad[..., POOL:POOL * (j_out + 1)].reshape(b, cin, j_out, POOL).transpose(0, 1, 3, 2)
    xe = jnp.concatenate([a, c], axis=2)
    xe = jnp.pad(xe, ((0, 0), (0, 0), (0, 0), (0, jpad - j_out)))
    return xe.reshape(b, cin * 2 * POOL, jpad)


def _stage0_kernel(b0_ref, w_ref, xe_ref, y_ref):
    c = jnp.dot(w_ref[...], xe_ref[0], preferred_element_type=jnp.float32,
                precision=jax.lax.Precision.HIGHEST)
    for co in range(2):
        pooled = jnp.max(c[POOL * co:POOL * (co + 1), :], axis=0, keepdims=True)
        y_ref[0, co:co + 1, :] = jnp.maximum(pooled + b0_ref[co], 0.0)


def _stage1_kernel(b1_ref, w_ref, xe_ref, wl_ref, bl_ref, out_ref):
    c = jnp.dot(w_ref[...], xe_ref[0], preferred_element_type=jnp.float32,
                precision=jax.lax.Precision.HIGHEST)
    acc = jnp.zeros((8, LANE), dtype=jnp.float32)
    for co in range(4):
        pooled = jnp.max(c[POOL * co:POOL * (co + 1), :], axis=0, keepdims=True)
        z = jnp.maximum(pooled + b1_ref[co], 0.0)
        lhs = jnp.broadcast_to(z, (8, LANE))
        acc = acc + jnp.dot(lhs, wl_ref[co], preferred_element_type=jnp.float32)
    out_ref[0] = acc + bl_ref[...]


def kernel(x, w0, b0, w1, b1, wl, bl):
    B, Cin, L = x.shape
    O = wl.shape[0]
    j0 = (L - KSZ + 1) // POOL
    j0pad = pl.cdiv(j0, LANE) * LANE
    j1 = (j0 - KSZ + 1) // POOL

    xe0 = _phase_major(x, j0, j0pad)
    t0 = _toeplitz(w0)

    grid0 = pltpu.PrefetchScalarGridSpec(
        num_scalar_prefetch=1,
        grid=(B,),
        in_specs=[
            pl.BlockSpec((2 * POOL, 2 * POOL), lambda bi, sm: (0, 0)),
            pl.BlockSpec((1, 2 * POOL, j0pad), lambda bi, sm: (bi, 0, 0)),
        ],
        out_specs=pl.BlockSpec((1, 2, j0pad), lambda bi, sm: (bi, 0, 0)),
    )
    y0 = pl.pallas_call(
        _stage0_kernel,
        out_shape=jax.ShapeDtypeStruct((B, 2, j0pad), jnp.float32),
        grid_spec=grid0,
        compiler_params=pltpu.CompilerParams(dimension_semantics=("parallel",)),
    )(b0, t0, xe0)

    xe1 = _phase_major(y0, j1, LANE)
    t1 = _toeplitz(w1)

    wl_r = wl.reshape(O, 4, j1).transpose(1, 2, 0)
    wl_r = jnp.pad(wl_r, ((0, 0), (0, LANE - j1), (0, LANE - O)))
    bl_p = jnp.pad(bl, (0, LANE - O)).reshape(1, LANE)

    grid1 = pltpu.PrefetchScalarGridSpec(
        num_scalar_prefetch=1,
        grid=(B,),
        in_specs=[
            pl.BlockSpec((4 * POOL, 4 * POOL), lambda bi, sm: (0, 0)),
            pl.BlockSpec((1, 4 * POOL, LANE), lambda bi, sm: (bi, 0, 0)),
            pl.BlockSpec((4, LANE, LANE), lambda bi, sm: (0, 0, 0)),
            pl.BlockSpec((1, LANE), lambda bi, sm: (0, 0)),
        ],
        out_specs=pl.BlockSpec((1, 8, LANE), lambda bi, sm: (bi, 0, 0)),
    )
    out = pl.pallas_call(
        _stage1_kernel,
        out_shape=jax.ShapeDtypeStruct((B, 8, LANE), jnp.float32),
        grid_spec=grid1,
        compiler_params=pltpu.CompilerParams(dimension_semantics=("parallel",)),
    )(b1, t1, xe1, wl_r, bl_p)
    return out[:, 0, :O]
